# baseline (speedup 1.0000x reference)
import functools

import jax
import jax.numpy as jnp
from jax import lax
from jax.experimental import pallas as pl
from jax.experimental.pallas import tpu as pltpu
from jax.experimental.pallas import tpu_sc as plsc

NC = 2
NS = 16
L = 16
NW = NC * NS
P = 8


def _emb_body(B, S, D, PW, ids_hbm, word_hbm, pos_hbm, type_hbm, gamma_hbm,
              beta_hbm, out_hbm, idx_v, gbuf0, gbuf1, obuf0, obuf1, pbuf0,
              pbuf1, type_v, gamma_v, beta_v, stat_v, gsem0, gsem1, osem0,
              osem1, isem):
    J = D // L
    NCHUNK = PW // P
    wid = lax.axis_index("s") * NC + lax.axis_index("c")
    pos0 = wid * PW

    id_handles = []
    for c in range(NCHUNK):
        for b in range(B):
            id_handles.append(pltpu.async_copy(
                ids_hbm.at[pl.ds(b * S + pos0 + c * P, P)],
                idx_v.at[c, pl.ds(b * P, P)], isem))
    pltpu.sync_copy(type_hbm.at[0], type_v)
    pltpu.sync_copy(gamma_hbm, gamma_v)
    pltpu.sync_copy(beta_hbm, beta_v)
    for hdl in id_handles:
        hdl.wait()

    gbufs = (gbuf0, gbuf1)
    obufs = (obuf0, obuf1)
    pbufs = (pbuf0, pbuf1)
    gsems = (gsem0, gsem1)
    osems = (osem0, osem1)

    def gather_desc(c, p):
        return pltpu.make_async_copy(
            word_hbm.at[idx_v.at[c]], gbufs[p], gsems[p])

    def pos_desc(c, p):
        return pltpu.make_async_copy(
            pos_hbm.at[pl.ds(pos0 + c * P, P)], pbufs[p], gsems[p])

    def out_descs(c, p):
        return [pltpu.make_async_copy(
            obufs[p].at[pl.ds(b * P, P)],
            out_hbm.at[pl.ds(b * S + pos0 + c * P, P)], osems[p])
            for b in range(B)]

    def start_chunk(c, p):
        gather_desc(c, p).start()
        pos_desc(c, p).start()

    def wait_chunk(c, p):
        gather_desc(c, p).wait()
        pos_desc(c, p).wait()

    def compute_chunk(c, p):
        gb = gbufs[p]
        ob = obufs[p]
        pb = pbufs[p]

        def stats(acc, acc2):
            s1 = jnp.broadcast_to(jnp.sum(acc), (L,))
            s2 = jnp.broadcast_to(jnp.sum(acc2), (L,))
            mean = s1 * (1.0 / D)
            var = s2 * (1.0 / D) - mean * mean
            x = var + 1e-12
            xi = lax.bitcast_convert_type(x, jnp.int32)
            yi = jnp.int32(0x5F3759DF) - lax.shift_right_logical(xi, 1)
            y = lax.bitcast_convert_type(yi, jnp.float32)
            hx = x * 0.5
            for _ in range(2):
                y = y * (1.5 - hx * y * y)
            return y, mean * y

        def p1body(pi, carry):
            acc = [None] * B
            acc2 = [None] * B
            for j in range(J):
                sl = pl.ds(j * L, L)
                pv = pb[pi, sl] + type_v[sl]
                for b in range(B):
                    v = gb[b * P + pi, sl] + pv
                    ob[b * P + pi, sl] = v
                    acc[b] = v if acc[b] is None else acc[b] + v
                    acc2[b] = v * v if acc2[b] is None else acc2[b] + v * v
            for b in range(B):
                y, ms = stats(acc[b], acc2[b])
                stat_v[0, b * P + pi, :] = y
                stat_v[1, b * P + pi, :] = ms
            return carry

        def p2body(pi, carry):
            ys = [stat_v[0, b * P + pi, :] for b in range(B)]
            mss = [stat_v[1, b * P + pi, :] for b in range(B)]
            for j in range(J):
                sl = pl.ds(j * L, L)
                g = gamma_v[sl]
                be = beta_v[sl]
                for b in range(B):
                    v = ob[b * P + pi, sl]
                    ob[b * P + pi, sl] = (v * ys[b] - mss[b]) * g + be
            return carry

        lax.fori_loop(0, P, p1body, 0)
        lax.fori_loop(0, P, p2body, 0)

    NI = NCHUNK // 2
    start_chunk(0, 0)
    start_chunk(1, 1)
    pltpu.async_copy(out_hbm.at[pl.ds(0, B * P)], obufs[0], osems[0])
    pltpu.async_copy(out_hbm.at[pl.ds(0, B * P)], obufs[1], osems[1])

    def steady(i, carry):
        c0 = 2 * i
        c1 = 2 * i + 1
        n0 = jnp.minimum(c0 + 2, NCHUNK - 1)
        n1 = jnp.minimum(c1 + 2, NCHUNK - 1)
        w0 = jnp.maximum(c0 - 2, 0)
        w1 = jnp.maximum(c1 - 2, 0)
        wait_chunk(c0, 0)
        for d in out_descs(w0, 0):
            d.wait()
        compute_chunk(c0, 0)
        for d in out_descs(c0, 0):
            d.start()
        start_chunk(n0, 0)
        wait_chunk(c1, 1)
        for d in out_descs(w1, 1):
            d.wait()
        compute_chunk(c1, 1)
        for d in out_descs(c1, 1):
            d.start()
        start_chunk(n1, 1)
        return carry

    lax.fori_loop(0, NI, steady, 0)

    cl0, cl1 = NCHUNK - 2, NCHUNK - 1
    wait_chunk(cl1, 0)
    wait_chunk(cl1, 1)
    for d in out_descs(cl0, 0):
        d.wait()
    for d in out_descs(cl1, 1):
        d.wait()


def kernel(input_ids, word_emb, pos_emb, type_emb, ln_gamma, ln_beta):
    B, S = input_ids.shape
    V, D = word_emb.shape
    assert S % NW == 0 and D % L == 0
    PW = S // NW

    mesh = plsc.VectorSubcoreMesh(
        core_axis_name="c", subcore_axis_name="s", num_cores=NC,
        num_subcores=NS)
    fn = pl.kernel(
        functools.partial(_emb_body, B, S, D, PW),
        out_type=jax.ShapeDtypeStruct((B * S, D), jnp.float32),
        mesh=mesh,
        compiler_params=pltpu.CompilerParams(needs_layout_passes=False),
        scratch_types=[
            pltpu.VMEM((PW // P, B * P), jnp.int32),
            pltpu.VMEM((B * P, D), jnp.float32),
            pltpu.VMEM((B * P, D), jnp.float32),
            pltpu.VMEM((B * P, D), jnp.float32),
            pltpu.VMEM((B * P, D), jnp.float32),
            pltpu.VMEM((P, D), jnp.float32),
            pltpu.VMEM((P, D), jnp.float32),
            pltpu.VMEM((D,), jnp.float32),
            pltpu.VMEM((D,), jnp.float32),
            pltpu.VMEM((D,), jnp.float32),
            pltpu.VMEM((2, B * P, L), jnp.float32),
            pltpu.SemaphoreType.DMA,
            pltpu.SemaphoreType.DMA,
            pltpu.SemaphoreType.DMA,
            pltpu.SemaphoreType.DMA,
            pltpu.SemaphoreType.DMA,
        ],
    )
    out = fn(input_ids.reshape(B * S), word_emb, pos_emb, type_emb,
             ln_gamma, ln_beta)
    return out.reshape(B, S, D)


# speedup vs baseline: 1.0245x; 1.0245x over previous
import functools

import jax
import jax.numpy as jnp
from jax import lax
from jax.experimental import pallas as pl
from jax.experimental.pallas import tpu as pltpu
from jax.experimental.pallas import tpu_sc as plsc

NC = 2
NS = 16
L = 16
NW = NC * NS
P = 8


def _emb_body(B, S, D, PW, ids_hbm, word_hbm, pos_hbm, type_hbm, gamma_hbm,
              beta_hbm, out_hbm, idx_v, gbuf0, gbuf1, obuf0, obuf1, pbuf0,
              pbuf1, type_v, gamma_v, beta_v, stat_v, gsem0, gsem1, osem0,
              osem1, isem):
    J = D // L
    NCHUNK = PW // P
    wid = lax.axis_index("s") * NC + lax.axis_index("c")
    pos0 = wid * PW

    id_handles = []
    for c in range(NCHUNK):
        for b in range(B):
            id_handles.append(pltpu.async_copy(
                ids_hbm.at[pl.ds(b * S + pos0 + c * P, P)],
                idx_v.at[c, pl.ds(b * P, P)], isem))
    pltpu.sync_copy(type_hbm.at[0], type_v)
    pltpu.sync_copy(gamma_hbm, gamma_v)
    pltpu.sync_copy(beta_hbm, beta_v)
    for hdl in id_handles:
        hdl.wait()

    gbufs = (gbuf0, gbuf1)
    obufs = (obuf0, obuf1)
    pbufs = (pbuf0, pbuf1)
    gsems = (gsem0, gsem1)
    osems = (osem0, osem1)

    def gather_desc(c, p):
        return pltpu.make_async_copy(
            word_hbm.at[idx_v.at[c]], gbufs[p], gsems[p])

    def pos_desc(c, p):
        return pltpu.make_async_copy(
            pos_hbm.at[pl.ds(pos0 + c * P, P)], pbufs[p], gsems[p])

    def out_descs(c, p):
        return [pltpu.make_async_copy(
            obufs[p].at[pl.ds(b * P, P)],
            out_hbm.at[pl.ds(b * S + pos0 + c * P, P)], osems[p])
            for b in range(B)]

    def start_chunk(c, p):
        gather_desc(c, p).start()
        pos_desc(c, p).start()

    def wait_chunk(c, p):
        gather_desc(c, p).wait()
        pos_desc(c, p).wait()

    def compute_chunk(c, p):
        gb = gbufs[p]
        ob = obufs[p]
        pb = pbufs[p]

        def stats(acc, acc2):
            s1 = jnp.broadcast_to(jnp.sum(acc), (L,))
            s2 = jnp.broadcast_to(jnp.sum(acc2), (L,))
            mean = s1 * (1.0 / D)
            var = s2 * (1.0 / D) - mean * mean
            x = var + 1e-12
            xi = lax.bitcast_convert_type(x, jnp.int32)
            yi = jnp.int32(0x5F3759DF) - lax.shift_right_logical(xi, 1)
            y = lax.bitcast_convert_type(yi, jnp.float32)
            hx = x * 0.5
            for _ in range(2):
                y = y * (1.5 - hx * y * y)
            return y, mean * y

        def p1body(pi, carry):
            acc = [None] * B
            acc2 = [None] * B
            for j in range(J):
                sl = pl.ds(j * L, L)
                pv = pb[pi, sl] + type_v[sl]
                for b in range(B):
                    v = gb[b * P + pi, sl] + pv
                    ob[b * P + pi, sl] = v
                    acc[b] = v if acc[b] is None else acc[b] + v
                    acc2[b] = v * v if acc2[b] is None else acc2[b] + v * v
            for b in range(B):
                y, ms = stats(acc[b], acc2[b])
                stat_v[0, b * P + pi, :] = y
                stat_v[1, b * P + pi, :] = ms
            return carry

        def p2body(pi, carry):
            ys = [stat_v[0, b * P + pi, :] for b in range(B)]
            mss = [stat_v[1, b * P + pi, :] for b in range(B)]
            for j in range(J):
                sl = pl.ds(j * L, L)
                g = gamma_v[sl]
                be = beta_v[sl]
                for b in range(B):
                    v = ob[b * P + pi, sl]
                    ob[b * P + pi, sl] = (v * ys[b] - mss[b]) * g + be
            return carry

        lax.fori_loop(0, P, p1body, 0)
        lax.fori_loop(0, P, p2body, 0)

    NI = NCHUNK // 2
    start_chunk(0, 0)
    start_chunk(1, 1)
    wait_chunk(0, 0)
    compute_chunk(0, 0)
    for d in out_descs(0, 0):
        d.start()
    start_chunk(2, 0)
    wait_chunk(1, 1)
    compute_chunk(1, 1)
    for d in out_descs(1, 1):
        d.start()
    start_chunk(3, 1)

    def steady(i, carry):
        c0 = 2 * i
        c1 = 2 * i + 1
        n0 = jnp.minimum(c0 + 2, NCHUNK - 1)
        n1 = jnp.minimum(c1 + 2, NCHUNK - 1)
        wait_chunk(c0, 0)
        for d in out_descs(c0 - 2, 0):
            d.wait()
        compute_chunk(c0, 0)
        for d in out_descs(c0, 0):
            d.start()
        start_chunk(n0, 0)
        wait_chunk(c1, 1)
        for d in out_descs(c1 - 2, 1):
            d.wait()
        compute_chunk(c1, 1)
        for d in out_descs(c1, 1):
            d.start()
        start_chunk(n1, 1)
        return carry

    lax.fori_loop(1, NI, steady, 0)

    cl0, cl1 = NCHUNK - 2, NCHUNK - 1
    wait_chunk(cl1, 0)
    wait_chunk(cl1, 1)
    for d in out_descs(cl0, 0):
        d.wait()
    for d in out_descs(cl1, 1):
        d.wait()


def kernel(input_ids, word_emb, pos_emb, type_emb, ln_gamma, ln_beta):
    B, S = input_ids.shape
    V, D = word_emb.shape
    assert S % NW == 0 and D % L == 0
    PW = S // NW

    mesh = plsc.VectorSubcoreMesh(
        core_axis_name="c", subcore_axis_name="s", num_cores=NC,
        num_subcores=NS)
    fn = pl.kernel(
        functools.partial(_emb_body, B, S, D, PW),
        out_type=jax.ShapeDtypeStruct((B * S, D), jnp.float32),
        mesh=mesh,
        compiler_params=pltpu.CompilerParams(needs_layout_passes=False),
        scratch_types=[
            pltpu.VMEM((PW // P, B * P), jnp.int32),
            pltpu.VMEM((B * P, D), jnp.float32),
            pltpu.VMEM((B * P, D), jnp.float32),
            pltpu.VMEM((B * P, D), jnp.float32),
            pltpu.VMEM((B * P, D), jnp.float32),
            pltpu.VMEM((P, D), jnp.float32),
            pltpu.VMEM((P, D), jnp.float32),
            pltpu.VMEM((D,), jnp.float32),
            pltpu.VMEM((D,), jnp.float32),
            pltpu.VMEM((D,), jnp.float32),
            pltpu.VMEM((2, B * P, L), jnp.float32),
            pltpu.SemaphoreType.DMA,
            pltpu.SemaphoreType.DMA,
            pltpu.SemaphoreType.DMA,
            pltpu.SemaphoreType.DMA,
            pltpu.SemaphoreType.DMA,
        ],
    )
    out = fn(input_ids.reshape(B * S), word_emb, pos_emb, type_emb,
             ln_gamma, ln_beta)
    return out.reshape(B, S, D)
